# baseline (speedup 1.0000x reference)
import jax
import jax.numpy as jnp
from jax.experimental import pallas as pl

PATCH = 16
EPS = 1e-5


def _embed_kernel(tok_ref, wg_ref, blin_ref, ln2g_ref, peh_ref, pew_ref, out_ref):
    n_tok_blk, pd = tok_ref.shape[1], tok_ref.shape[2]
    x = tok_ref[0]
    m = jnp.mean(x, axis=-1, keepdims=True)
    v = jnp.mean(x * x, axis=-1, keepdims=True) - m * m
    rs = jax.lax.rsqrt(v + EPS)
    xn = (x * rs - m * rs).astype(jnp.bfloat16)
    y = jnp.dot(xn, wg_ref[...], preferred_element_type=jnp.float32)
    y = y + blin_ref[0]
    m2 = jnp.mean(y, axis=-1, keepdims=True)
    v2 = jnp.mean(y * y, axis=-1, keepdims=True) - m2 * m2
    rs2 = jax.lax.rsqrt(v2 + EPS)
    yn = y * rs2 - m2 * rs2
    pos = peh_ref[...][:, None, :] + pew_ref[...][None, :, :]
    out_ref[0] = yn * ln2g_ref[0] + pos.reshape(n_tok_blk, out_ref.shape[2])


def kernel(imgs, ln1_g, W_lin, b_lin, ln2_g, pos_embed_height, pos_embed_width):
    B, C, H, W = imgs.shape
    P = PATCH
    h, w = H // P, W // P
    dim, patch_dim = W_lin.shape
    n_tok = h * w

    wg = (W_lin * ln1_g).T.astype(jnp.bfloat16)
    blin = b_lin.reshape(1, dim)
    ln2g = ln2_g.reshape(1, dim)

    CH = 4
    SPLIT = 2
    hs = h // SPLIT

    chunks = []
    for s in range(0, B, CH):
        tok = imgs[s:s + CH].reshape(CH, C, h, P, w, P)
        tok = tok.transpose(0, 2, 4, 1, 3, 5).reshape(CH, n_tok, patch_dim)
        out_c = pl.pallas_call(
            _embed_kernel,
            grid=(CH, SPLIT),
            in_specs=[
                pl.BlockSpec((1, hs * w, patch_dim), lambda i, j: (i, j, 0)),
                pl.BlockSpec((patch_dim, dim), lambda i, j: (0, 0)),
                pl.BlockSpec((1, dim), lambda i, j: (0, 0)),
                pl.BlockSpec((1, dim), lambda i, j: (0, 0)),
                pl.BlockSpec((hs, dim), lambda i, j: (j, 0)),
                pl.BlockSpec((w, dim), lambda i, j: (0, 0)),
            ],
            out_specs=pl.BlockSpec((1, hs * w, dim), lambda i, j: (i, j, 0)),
            out_shape=jax.ShapeDtypeStruct((CH, n_tok, dim), jnp.float32),
        )(tok, wg, blin, ln2g, pos_embed_height, pos_embed_width)
        chunks.append(out_c)
    return jnp.concatenate(chunks, axis=0)


# speedup vs baseline: 2.3417x; 2.3417x over previous
import jax
import jax.numpy as jnp
from jax.experimental import pallas as pl
from jax.experimental.pallas import tpu as pltpu

PATCH = 16
EPS = 1e-5


def _embed_kernel(img_ref, wg_ref, blin_ref, ln2g_ref, peh_ref, pew_ref, out_ref):
    C, Hs, W = img_ref.shape[1:]
    P = PATCH
    h, w = Hs // P, W // P
    pd = C * P * P
    im = img_ref[0]
    xb = im.astype(jnp.bfloat16)
    xb = xb.reshape(C, h, P, w, P).transpose(1, 3, 0, 2, 4).reshape(h * w, pd)
    x = xb.astype(jnp.float32)
    m = jnp.mean(x, axis=-1, keepdims=True)
    v = jnp.mean(x * x, axis=-1, keepdims=True) - m * m
    rs = jax.lax.rsqrt(v + EPS)
    xn = (x * rs - m * rs).astype(jnp.bfloat16)
    y = jnp.dot(xn, wg_ref[...], preferred_element_type=jnp.float32)
    y = y + blin_ref[0]
    m2 = jnp.mean(y, axis=-1, keepdims=True)
    v2 = jnp.mean(y * y, axis=-1, keepdims=True) - m2 * m2
    rs2 = jax.lax.rsqrt(v2 + EPS)
    yn = y * rs2 - m2 * rs2
    pos = peh_ref[...][:, None, :] + pew_ref[...][None, :, :]
    out_ref[0] = yn * ln2g_ref[0] + pos.reshape(h * w, out_ref.shape[2])


def kernel(imgs, ln1_g, W_lin, b_lin, ln2_g, pos_embed_height, pos_embed_width):
    B, C, H, W = imgs.shape
    P = PATCH
    h, w = H // P, W // P
    dim, patch_dim = W_lin.shape
    n_tok = h * w

    wg = (W_lin * ln1_g).T.astype(jnp.bfloat16)

    SPLIT = 4
    hs = h // SPLIT

    out = pl.pallas_call(
        _embed_kernel,
        grid=(B, SPLIT),
        in_specs=[
            pl.BlockSpec((1, C, H // SPLIT, W), lambda i, j: (i, 0, j, 0)),
            pl.BlockSpec((patch_dim, dim), lambda i, j: (0, 0)),
            pl.BlockSpec((1, dim), lambda i, j: (0, 0)),
            pl.BlockSpec((1, dim), lambda i, j: (0, 0)),
            pl.BlockSpec((hs, dim), lambda i, j: (j, 0)),
            pl.BlockSpec((w, dim), lambda i, j: (0, 0)),
        ],
        out_specs=pl.BlockSpec((1, hs * w, dim), lambda i, j: (i, j, 0)),
        out_shape=jax.ShapeDtypeStruct((B, n_tok, dim), jnp.float32),
        compiler_params=pltpu.CompilerParams(
            dimension_semantics=("parallel", "arbitrary")),
    )(imgs, wg, b_lin.reshape(1, dim), ln2_g.reshape(1, dim),
      pos_embed_height, pos_embed_width)
    return out


# speedup vs baseline: 2.3713x; 1.0126x over previous
import jax
import jax.numpy as jnp
from jax.experimental import pallas as pl
from jax.experimental.pallas import tpu as pltpu

PATCH = 16
EPS = 1e-5


def _embed_kernel(img_ref, wg_ref, blin_ref, ln2g_ref, peh_ref, pew_ref, out_ref):
    C, Hs, W = img_ref.shape[1:]
    P = PATCH
    h, w = Hs // P, W // P
    pd = C * P * P
    im = img_ref[0]
    xb = im.astype(jnp.bfloat16)
    xb = xb.reshape(C, h, P, w, P).transpose(1, 3, 0, 2, 4).reshape(h * w, pd)
    x = xb.astype(jnp.float32)
    m = jnp.mean(x, axis=-1, keepdims=True)
    v = jnp.mean(x * x, axis=-1, keepdims=True) - m * m
    rs = jax.lax.rsqrt(v + EPS)
    xn = (x * rs - m * rs).astype(jnp.bfloat16)
    y = jnp.dot(xn, wg_ref[...], preferred_element_type=jnp.float32)
    y = y + blin_ref[0]
    m2 = jnp.mean(y, axis=-1, keepdims=True)
    v2 = jnp.mean(y * y, axis=-1, keepdims=True) - m2 * m2
    rs2 = jax.lax.rsqrt(v2 + EPS)
    yn = y * rs2 - m2 * rs2
    pos = peh_ref[...][:, None, :] + pew_ref[...][None, :, :]
    out_ref[0] = yn * ln2g_ref[0] + pos.reshape(h * w, out_ref.shape[2])


def kernel(imgs, ln1_g, W_lin, b_lin, ln2_g, pos_embed_height, pos_embed_width):
    B, C, H, W = imgs.shape
    P = PATCH
    h, w = H // P, W // P
    dim, patch_dim = W_lin.shape
    n_tok = h * w

    wg = (W_lin * ln1_g).T.astype(jnp.bfloat16)

    SPLIT = 2
    hs = h // SPLIT

    out = pl.pallas_call(
        _embed_kernel,
        grid=(B, SPLIT),
        in_specs=[
            pl.BlockSpec((1, C, H // SPLIT, W), lambda i, j: (i, 0, j, 0)),
            pl.BlockSpec((patch_dim, dim), lambda i, j: (0, 0)),
            pl.BlockSpec((1, dim), lambda i, j: (0, 0)),
            pl.BlockSpec((1, dim), lambda i, j: (0, 0)),
            pl.BlockSpec((hs, dim), lambda i, j: (j, 0)),
            pl.BlockSpec((w, dim), lambda i, j: (0, 0)),
        ],
        out_specs=pl.BlockSpec((1, hs * w, dim), lambda i, j: (i, j, 0)),
        out_shape=jax.ShapeDtypeStruct((B, n_tok, dim), jnp.float32),
        compiler_params=pltpu.CompilerParams(
            dimension_semantics=("parallel", "arbitrary")),
    )(imgs, wg, b_lin.reshape(1, dim), ln2_g.reshape(1, dim),
      pos_embed_height, pos_embed_width)
    return out


# speedup vs baseline: 2.4343x; 1.0266x over previous
import jax
import jax.numpy as jnp
from jax.experimental import pallas as pl
from jax.experimental.pallas import tpu as pltpu

PATCH = 16
EPS = 1e-5


def _embed_kernel(img_ref, wg_ref, blin_ref, ln2g_ref, peh_ref, pew_ref, out_ref):
    C, Hs, W = img_ref.shape[1:]
    P = PATCH
    h, w = Hs // P, W // P
    pd = C * P * P
    im = img_ref[0]
    xb = im.astype(jnp.bfloat16)
    xb = xb.reshape(C, h, P, w, P).transpose(1, 3, 0, 2, 4).reshape(h * w, pd)
    x = xb.astype(jnp.float32)
    m = jnp.mean(x, axis=-1, keepdims=True)
    v = jnp.mean(x * x, axis=-1, keepdims=True) - m * m
    rs = jax.lax.rsqrt(v + EPS)
    xn = (x * rs - m * rs).astype(jnp.bfloat16)
    y = jnp.dot(xn, wg_ref[...], preferred_element_type=jnp.float32)
    y = y + blin_ref[0]
    m2 = jnp.mean(y, axis=-1, keepdims=True)
    v2 = jnp.mean(y * y, axis=-1, keepdims=True) - m2 * m2
    rs2 = jax.lax.rsqrt(v2 + EPS)
    yn = y * rs2 - m2 * rs2
    pos = peh_ref[...][:, None, :] + pew_ref[...][None, :, :]
    out_ref[0] = yn * ln2g_ref[0] + pos.reshape(h * w, out_ref.shape[2])


def kernel(imgs, ln1_g, W_lin, b_lin, ln2_g, pos_embed_height, pos_embed_width):
    B, C, H, W = imgs.shape
    P = PATCH
    h, w = H // P, W // P
    dim, patch_dim = W_lin.shape
    n_tok = h * w

    wg = (W_lin * ln1_g).T.astype(jnp.bfloat16)

    SPLIT = 1
    hs = h // SPLIT

    out = pl.pallas_call(
        _embed_kernel,
        grid=(B, SPLIT),
        in_specs=[
            pl.BlockSpec((1, C, H // SPLIT, W), lambda i, j: (i, 0, j, 0)),
            pl.BlockSpec((patch_dim, dim), lambda i, j: (0, 0)),
            pl.BlockSpec((1, dim), lambda i, j: (0, 0)),
            pl.BlockSpec((1, dim), lambda i, j: (0, 0)),
            pl.BlockSpec((hs, dim), lambda i, j: (j, 0)),
            pl.BlockSpec((w, dim), lambda i, j: (0, 0)),
        ],
        out_specs=pl.BlockSpec((1, hs * w, dim), lambda i, j: (i, j, 0)),
        out_shape=jax.ShapeDtypeStruct((B, n_tok, dim), jnp.float32),
        compiler_params=pltpu.CompilerParams(
            dimension_semantics=("parallel", "arbitrary")),
    )(imgs, wg, b_lin.reshape(1, dim), ln2_g.reshape(1, dim),
      pos_embed_height, pos_embed_width)
    return out
